# baseline (speedup 1.0000x reference)
import functools
import math

import jax
import jax.numpy as jnp
from jax.experimental import pallas as pl
from jax.experimental.pallas import tpu as pltpu

_INV_SQRT2 = 1.0 / math.sqrt(2.0)


def _gelu_exact_f32(h):
    return 0.5 * h * (1.0 + jax.lax.erf(h * jnp.float32(_INV_SQRT2)))


def _ffn_kernel(x_ref, w1_ref, b1_ref, w2_ref, b2_ref, o_ref, *, subtiles):
    tm = x_ref.shape[0]
    sub = tm // subtiles
    for s in range(subtiles):
        rows = pl.ds(s * sub, sub)
        h = jnp.dot(x_ref[rows, :], w1_ref[...],
                    preferred_element_type=jnp.float32)
        g = _gelu_exact_f32(h + b1_ref[...])
        o_ref[rows, :] = (jnp.dot(g.astype(jnp.bfloat16), w2_ref[...],
                                  preferred_element_type=jnp.float32)
                          + b2_ref[...])


@functools.partial(jax.jit, static_argnames=("tm", "subtiles"))
def _mlp_forward(x, w1, b1, w2, b2, *, tm=512, subtiles=2):
    B, N, in_feat = x.shape
    hid = w1.shape[1]
    out_feat = w2.shape[1]
    M = B * N
    x2 = x.reshape(M, in_feat).astype(jnp.bfloat16)
    w1b = w1.astype(jnp.bfloat16)
    w2b = w2.astype(jnp.bfloat16)
    b1_2d = b1.reshape(1, hid)
    b2_2d = b2.reshape(1, out_feat)
    single = pl.Buffered(1)

    cost = pl.CostEstimate(
        flops=int(2 * M * (in_feat * hid + hid * out_feat)),
        transcendentals=int(M * hid),
        bytes_accessed=int(M * in_feat * 2
                           + (in_feat * hid + hid * out_feat) * 2
                           + (hid + out_feat) * 4
                           + M * out_feat * 4),
    )

    y2 = pl.pallas_call(
        functools.partial(_ffn_kernel, subtiles=subtiles),
        out_shape=jax.ShapeDtypeStruct((M, out_feat), jnp.float32),
        grid_spec=pltpu.PrefetchScalarGridSpec(
            num_scalar_prefetch=0,
            grid=(pl.cdiv(M, tm),),
            in_specs=[
                pl.BlockSpec((tm, in_feat), lambda i: (i, 0)),
                pl.BlockSpec((in_feat, hid), lambda i: (0, 0),
                             pipeline_mode=single),
                pl.BlockSpec((1, hid), lambda i: (0, 0), pipeline_mode=single),
                pl.BlockSpec((hid, out_feat), lambda i: (0, 0),
                             pipeline_mode=single),
                pl.BlockSpec((1, out_feat), lambda i: (0, 0),
                             pipeline_mode=single),
            ],
            out_specs=pl.BlockSpec((tm, out_feat), lambda i: (i, 0)),
        ),
        compiler_params=pltpu.CompilerParams(
            dimension_semantics=("parallel",),
            vmem_limit_bytes=52 * 1024 * 1024,
        ),
        cost_estimate=cost,
    )(x2, w1b, b1_2d, w2b, b2_2d)

    return y2.reshape(B, N, out_feat)


def kernel(x, w1, b1, w2, b2):
    return _mlp_forward(x, w1, b1, w2, b2)


# speedup vs baseline: 1.2417x; 1.2417x over previous
import functools
import math

import jax
import jax.numpy as jnp
from jax.experimental import pallas as pl
from jax.experimental.pallas import tpu as pltpu

_INV_SQRT2 = 1.0 / math.sqrt(2.0)


def _gelu_exact_f32(h):
    return 0.5 * h * (1.0 + jax.lax.erf(h * jnp.float32(_INV_SQRT2)))


def _ffn_kernel(x_ref, w1_ref, b1_ref, w2_ref, b2_ref, o_ref):
    h = jnp.dot(x_ref[...], w1_ref[...], preferred_element_type=jnp.float32)
    g = _gelu_exact_f32(h + b1_ref[...])
    y = jnp.dot(g, w2_ref[...], preferred_element_type=jnp.float32)
    o_ref[...] = y + b2_ref[...]


@functools.partial(jax.jit, static_argnames=("tm",))
def _mlp_forward(x, w1, b1, w2, b2, *, tm=256):
    B, N, in_feat = x.shape
    hid = w1.shape[1]
    out_feat = w2.shape[1]
    M = B * N
    x2 = x.reshape(M, in_feat)
    b1_2d = b1.reshape(1, hid)
    b2_2d = b2.reshape(1, out_feat)
    single = pl.Buffered(1)

    cost = pl.CostEstimate(
        flops=int(2 * M * (in_feat * hid + hid * out_feat)),
        transcendentals=int(M * hid),
        bytes_accessed=int(M * in_feat * 4
                           + (in_feat * hid + hid + hid * out_feat + out_feat) * 4
                           + M * out_feat * 4),
    )

    y2 = pl.pallas_call(
        _ffn_kernel,
        out_shape=jax.ShapeDtypeStruct((M, out_feat), jnp.float32),
        grid_spec=pltpu.PrefetchScalarGridSpec(
            num_scalar_prefetch=0,
            grid=(pl.cdiv(M, tm),),
            in_specs=[
                pl.BlockSpec((tm, in_feat), lambda i: (i, 0)),
                pl.BlockSpec((in_feat, hid), lambda i: (0, 0),
                             pipeline_mode=single),
                pl.BlockSpec((1, hid), lambda i: (0, 0), pipeline_mode=single),
                pl.BlockSpec((hid, out_feat), lambda i: (0, 0),
                             pipeline_mode=single),
                pl.BlockSpec((1, out_feat), lambda i: (0, 0),
                             pipeline_mode=single),
            ],
            out_specs=pl.BlockSpec((tm, out_feat), lambda i: (i, 0)),
        ),
        compiler_params=pltpu.CompilerParams(
            dimension_semantics=("parallel",),
            vmem_limit_bytes=96 * 1024 * 1024,
        ),
        cost_estimate=cost,
    )(x2, w1, b1_2d, w2, b2_2d)

    return y2.reshape(B, N, out_feat)


def kernel(x, w1, b1, w2, b2):
    return _mlp_forward(x, w1, b1, w2, b2)


# speedup vs baseline: 1.2978x; 1.0452x over previous
import functools
import math

import jax
import jax.numpy as jnp
from jax.experimental import pallas as pl
from jax.experimental.pallas import tpu as pltpu

_INV_SQRT2 = 1.0 / math.sqrt(2.0)


def _gelu_exact_f32(h):
    return 0.5 * h * (1.0 + jax.lax.erf(h * jnp.float32(_INV_SQRT2)))


def _ffn_kernel(x_ref, w1_ref, b1_ref, w2_ref, b2_ref, o_ref):
    h = jnp.dot(x_ref[...], w1_ref[...], preferred_element_type=jnp.float32)
    g = _gelu_exact_f32(h + b1_ref[...])
    y = jnp.dot(g, w2_ref[...], preferred_element_type=jnp.float32)
    o_ref[...] = y + b2_ref[...]


@functools.partial(jax.jit, static_argnames=("tm",))
def _mlp_forward(x, w1, b1, w2, b2, *, tm=1024):
    B, N, in_feat = x.shape
    hid = w1.shape[1]
    out_feat = w2.shape[1]
    M = B * N
    x2 = x.reshape(M, in_feat)
    b1_2d = b1.reshape(1, hid)
    b2_2d = b2.reshape(1, out_feat)
    single = pl.Buffered(1)

    cost = pl.CostEstimate(
        flops=int(2 * M * (in_feat * hid + hid * out_feat)),
        transcendentals=int(M * hid),
        bytes_accessed=int(M * in_feat * 4
                           + (in_feat * hid + hid + hid * out_feat + out_feat) * 4
                           + M * out_feat * 4),
    )

    y2 = pl.pallas_call(
        _ffn_kernel,
        out_shape=jax.ShapeDtypeStruct((M, out_feat), jnp.float32),
        grid_spec=pltpu.PrefetchScalarGridSpec(
            num_scalar_prefetch=0,
            grid=(pl.cdiv(M, tm),),
            in_specs=[
                pl.BlockSpec((tm, in_feat), lambda i: (i, 0)),
                pl.BlockSpec((in_feat, hid), lambda i: (0, 0),
                             pipeline_mode=single),
                pl.BlockSpec((1, hid), lambda i: (0, 0), pipeline_mode=single),
                pl.BlockSpec((hid, out_feat), lambda i: (0, 0),
                             pipeline_mode=single),
                pl.BlockSpec((1, out_feat), lambda i: (0, 0),
                             pipeline_mode=single),
            ],
            out_specs=pl.BlockSpec((tm, out_feat), lambda i: (i, 0)),
        ),
        compiler_params=pltpu.CompilerParams(
            dimension_semantics=("parallel",),
            vmem_limit_bytes=96 * 1024 * 1024,
        ),
        cost_estimate=cost,
    )(x2, w1, b1_2d, w2, b2_2d)

    return y2.reshape(B, N, out_feat)


def kernel(x, w1, b1, w2, b2):
    return _mlp_forward(x, w1, b1, w2, b2)
